# baseline (speedup 1.0000x reference)
import dataclasses
import functools

import jax
import jax.numpy as jnp
from jax import lax
from jax.experimental import pallas as pl
from jax.experimental.pallas import tpu as pltpu
from jax.experimental.pallas import tpu_sc as plsc

N = 10000
E = 320000
D = 128

NC = 2
NS = 16
L = 16
NW = NC * NS

EXT = E + N
EPT = 10368
EPAD = EPT * NW
CW = 576
NCHW = EPT // CW
C = 96
NCH = EPT // C
SB = 624
ET = N - NS * SB


def _prep_body(x_ref, w_ref, asrc_ref, adst_ref, h_ref, as_ref, ad_ref, mv_ref):
    x = x_ref[...]
    w = w_ref[...]
    h = lax.dot_general(x, w, (((1,), (1,)), ((), ())),
                        preferred_element_type=jnp.float32)
    h_ref[...] = h
    asr = lax.dot_general(asrc_ref[...], h, (((1,), (1,)), ((), ())),
                          preferred_element_type=jnp.float32)
    adr = lax.dot_general(adst_ref[...], h, (((1,), (1,)), ((), ())),
                          preferred_element_type=jnp.float32)
    as_ref[...] = asr
    ad_ref[...] = adr
    m = jnp.max(asr) + jnp.max(adr)
    m = jnp.maximum(m, 0.2 * m)
    mv_ref[...] = jnp.full((8, 128), m, jnp.float32)


_prep = pl.pallas_call(
    _prep_body,
    out_shape=[
        jax.ShapeDtypeStruct((N, D), jnp.float32),
        jax.ShapeDtypeStruct((1, N), jnp.float32),
        jax.ShapeDtypeStruct((1, N), jnp.float32),
        jax.ShapeDtypeStruct((8, 128), jnp.float32),
    ],
)


def _fin_body(p_ref, den_ref, b_ref, o_ref):
    num = p_ref[0] + p_ref[1]
    ones = jnp.ones((NW, 1), jnp.float32)
    dcol = lax.dot_general(den_ref[...], ones, (((0,), (0,)), ((), ())),
                           preferred_element_type=jnp.float32)
    o = num / (dcol + 1e-16) + b_ref[...]
    o_ref[...] = jnp.where(o > 0, o, jnp.exp(jnp.minimum(o, 0.0)) - 1.0)


_fin = pl.pallas_call(
    _fin_body,
    out_shape=jax.ShapeDtypeStruct((N, D), jnp.float32),
)


_sc_params = pltpu.CompilerParams()
if "needs_layout_passes" in pltpu.CompilerParams.__dataclass_fields__:
    _sc_params = dataclasses.replace(_sc_params, needs_layout_passes=False)

_MESH = plsc.VectorSubcoreMesh(core_axis_name="c", subcore_axis_name="s")


def _wpass_body(src_hbm, dst_hbm, as_hbm, ad_hbm, mv_hbm,
                w_hbm, denp_hbm,
                as_v, ad_v, mv_v, src0_v, src1_v, dst0_v, dst1_v,
                w0_v, w1_v, den_v, si0, si1, so0, so1):
    cid = lax.axis_index("c")
    sid = lax.axis_index("s")
    wid = cid * NS + sid
    srcs, dsts = [src0_v, src1_v], [dst0_v, dst1_v]
    wouts, si, so = [w0_v, w1_v], [si0, si1], [so0, so1]

    pltpu.sync_copy(as_hbm.at[0], as_v)
    pltpu.sync_copy(ad_hbm.at[0], ad_v)
    pltpu.sync_copy(mv_hbm.at[0, pl.ds(0, L)], mv_v)

    zero16 = jnp.zeros((L,), jnp.float32)

    @pl.loop(0, N // L)
    def _zden(r):
        den_v[pl.ds(r * L, L)] = zero16

    mvec = mv_v[...]

    def _issue_idx(c, p):
        base = wid * EPT + c * CW
        pltpu.async_copy(src_hbm.at[pl.ds(base, CW)], srcs[p], si[p])
        pltpu.async_copy(dst_hbm.at[pl.ds(base, CW)], dsts[p], si[p])

    def _wait_idx(p):
        pltpu.make_async_copy(src_hbm.at[pl.ds(0, CW)], srcs[p], si[p]).wait()
        pltpu.make_async_copy(dst_hbm.at[pl.ds(0, CW)], dsts[p], si[p]).wait()

    def _body(c, p, pf_idx, wait_out):
        _wait_idx(p)
        if wait_out:
            pltpu.make_async_copy(w_hbm.at[pl.ds(0, CW)], wouts[p],
                                  so[p]).wait()
        for g in range(CW // L):
            s16 = srcs[p][pl.ds(g * L, L)]
            d16 = dsts[p][pl.ds(g * L, L)]
            av = plsc.load_gather(as_v, [s16])
            dv = plsc.load_gather(ad_v, [d16])
            t = av + dv
            alpha = jnp.maximum(t, 0.2 * t)
            wv = jnp.exp(alpha - mvec)
            eid = wid * EPT + c * CW + g * L + lax.iota(jnp.int32, L)
            wv = jnp.where(eid < EXT, wv, 0.0)
            wouts[p][pl.ds(g * L, L)] = wv
            plsc.addupdate_scatter(den_v, [d16], wv)
        base = wid * EPT + c * CW
        pltpu.async_copy(wouts[p], w_hbm.at[pl.ds(base, CW)], so[p])
        if pf_idx:
            _issue_idx(c + 2, p)

    _issue_idx(0, 0)
    _issue_idx(1, 1)
    _body(0, 0, pf_idx=True, wait_out=False)
    _body(1, 1, pf_idx=True, wait_out=False)

    @pl.loop(0, (NCHW - 4) // 2)
    def _main(j):
        for b in range(2):
            _body(2 + 2 * j + b, b, pf_idx=True, wait_out=True)

    _body(NCHW - 2, 0, pf_idx=False, wait_out=True)
    _body(NCHW - 1, 1, pf_idx=False, wait_out=True)
    pltpu.make_async_copy(w_hbm.at[pl.ds(0, CW)], wouts[0], so[0]).wait()
    pltpu.make_async_copy(w_hbm.at[pl.ds(0, CW)], wouts[1], so[1]).wait()

    pltpu.sync_copy(den_v, denp_hbm.at[wid])


_wpass = pl.kernel(
    _wpass_body,
    out_type=[
        jax.ShapeDtypeStruct((EPAD,), jnp.float32),
        jax.ShapeDtypeStruct((NW, N), jnp.float32),
    ],
    mesh=_MESH,
    compiler_params=_sc_params,
    scratch_types=[
        pltpu.VMEM((N,), jnp.float32),
        pltpu.VMEM((N,), jnp.float32),
        pltpu.VMEM((L,), jnp.float32),
        pltpu.VMEM((CW,), jnp.int32),
        pltpu.VMEM((CW,), jnp.int32),
        pltpu.VMEM((CW,), jnp.int32),
        pltpu.VMEM((CW,), jnp.int32),
        pltpu.VMEM((CW,), jnp.float32),
        pltpu.VMEM((CW,), jnp.float32),
        pltpu.VMEM((N,), jnp.float32),
        pltpu.SemaphoreType.DMA,
        pltpu.SemaphoreType.DMA,
        pltpu.SemaphoreType.DMA,
        pltpu.SemaphoreType.DMA,
    ],
)


def _rpass_body(h_hbm, src_hbm, dst_hbm, w_hbm,
                accp_hbm,
                src_v, dst_v, w_v, rows0_v, rows1_v, rows2_v,
                acc_sh, si0, si1, si2, si3, si4, si5,
                sg0, sg1, sg2, sc0, sc1, sc2):
    cid = lax.axis_index("c")
    sid = lax.axis_index("s")
    wid = cid * NS + sid
    rows = [rows0_v, rows1_v, rows2_v]
    si = [si0, si1, si2, si3, si4, si5]
    sg, sc = [sg0, sg1, sg2], [sc0, sc1, sc2]

    zero16 = jnp.zeros((L,), jnp.float32)

    @pl.loop(0, C)
    def _zrow(r):
        for k in range(D // L):
            rows0_v[r, pl.ds(k * L, L)] = zero16

    r0 = sid * SB

    @pl.loop(0, SB // C)
    def _zacc(j):
        pltpu.sync_copy(rows0_v, acc_sh.at[pl.ds(r0 + j * C, C)])

    zrem = SB % C
    if zrem:
        pltpu.sync_copy(rows0_v.at[pl.ds(0, zrem)],
                        acc_sh.at[pl.ds(r0 + (SB // C) * C, zrem)])

    @pl.when(sid == NS - 1)
    def _ztail():
        pltpu.sync_copy(rows0_v.at[pl.ds(0, ET)], acc_sh.at[pl.ds(NS * SB, ET)])

    plsc.subcore_barrier()

    def _issue_idx(c, p):
        base = wid * EPT + c * C
        pltpu.async_copy(src_hbm.at[pl.ds(base, C)], src_v.at[p], si[p])
        pltpu.async_copy(dst_hbm.at[pl.ds(base, C)], dst_v.at[p], si[p])
        pltpu.async_copy(w_hbm.at[pl.ds(base, C)], w_v.at[p], si[p])

    def _wait_idx(p):
        pltpu.make_async_copy(src_hbm.at[pl.ds(0, C)], src_v.at[p],
                              si[p]).wait()
        pltpu.make_async_copy(dst_hbm.at[pl.ds(0, C)], dst_v.at[p],
                              si[p]).wait()
        pltpu.make_async_copy(w_hbm.at[pl.ds(0, C)], w_v.at[p], si[p]).wait()

    H2 = C // 2

    def _issue_gat(p6, p3):
        pltpu.async_copy(h_hbm.at[src_v.at[p6, pl.ds(0, H2)]],
                         rows[p3].at[pl.ds(0, H2)], sg[p3])
        pltpu.async_copy(h_hbm.at[src_v.at[p6, pl.ds(H2, H2)]],
                         rows[p3].at[pl.ds(H2, H2)], sg[p3])

    def _wait_gat(p6, p3):
        pltpu.make_async_copy(h_hbm.at[src_v.at[p6, pl.ds(0, H2)]],
                              rows[p3].at[pl.ds(0, H2)], sg[p3]).wait()
        pltpu.make_async_copy(h_hbm.at[src_v.at[p6, pl.ds(H2, H2)]],
                              rows[p3].at[pl.ds(H2, H2)], sg[p3]).wait()

    def _wait_sc(p6, p3):
        pltpu.make_async_copy(rows[p3], acc_sh.at[dst_v.at[p6]], sc[p3]).wait()

    def _body(c, p6, p3, wait_sc, pf_gat, pf_idx):
        _wait_gat(p6, p3)

        @pl.loop(0, C)
        def _scale(e):
            s = plsc.load_gather(w_v.at[p6], [jnp.full((L,), e, jnp.int32)])
            for k in range(D // L):
                rows[p3][e, pl.ds(k * L, L)] = rows[p3][e, pl.ds(k * L, L)] * s

        if wait_sc:
            _wait_sc((p6 + 4) % 6, (p3 + 1) % 3)
        pltpu.async_copy(rows[p3], acc_sh.at[dst_v.at[p6]], sc[p3], add=True)
        if pf_gat:
            _wait_idx((p6 + 1) % 6)
            _issue_gat((p6 + 1) % 6, (p3 + 1) % 3)
        if pf_idx:
            _issue_idx(c + 3, (p6 + 3) % 6)

    _issue_idx(0, 0)
    _issue_idx(1, 1)
    _issue_idx(2, 2)
    _wait_idx(0)
    _issue_gat(0, 0)
    _body(0, 0, 0, wait_sc=False, pf_gat=True, pf_idx=True)
    _body(1, 1, 1, wait_sc=False, pf_gat=True, pf_idx=True)

    @pl.loop(0, (NCH - 6) // 6)
    def _main(j):
        for u in range(6):
            c = 2 + 6 * j + u
            _body(c, (2 + u) % 6, (2 + u) % 3,
                  wait_sc=True, pf_gat=True, pf_idx=True)

    for u in range(4):
        c = NCH - 4 + u
        _body(c, c % 6, c % 3, wait_sc=True,
              pf_gat=(u < 3), pf_idx=(u == 0))

    _wait_sc((NCH - 2) % 6, (NCH - 2) % 3)
    _wait_sc((NCH - 1) % 6, (NCH - 1) % 3)

    plsc.subcore_barrier()
    pltpu.sync_copy(acc_sh.at[pl.ds(r0, SB)], accp_hbm.at[cid, pl.ds(r0, SB)])

    @pl.when(sid == NS - 1)
    def _dtail():
        t0 = NS * SB
        pltpu.sync_copy(acc_sh.at[pl.ds(t0, ET)], accp_hbm.at[cid, pl.ds(t0, ET)])


_rpass = pl.kernel(
    _rpass_body,
    out_type=jax.ShapeDtypeStruct((NC, N, D), jnp.float32),
    mesh=_MESH,
    compiler_params=_sc_params,
    scratch_types=[
        pltpu.VMEM((6, C), jnp.int32),
        pltpu.VMEM((6, C), jnp.int32),
        pltpu.VMEM((6, C), jnp.float32),
        pltpu.VMEM((C, D), jnp.float32),
        pltpu.VMEM((C, D), jnp.float32),
        pltpu.VMEM((C, D), jnp.float32),
        pltpu.VMEM_SHARED((N, D), jnp.float32),
        pltpu.SemaphoreType.DMA,
        pltpu.SemaphoreType.DMA,
        pltpu.SemaphoreType.DMA,
        pltpu.SemaphoreType.DMA,
        pltpu.SemaphoreType.DMA,
        pltpu.SemaphoreType.DMA,
        pltpu.SemaphoreType.DMA,
        pltpu.SemaphoreType.DMA,
        pltpu.SemaphoreType.DMA,
        pltpu.SemaphoreType.DMA,
        pltpu.SemaphoreType.DMA,
        pltpu.SemaphoreType.DMA,
    ],
)


def _gat_layer(x, src, dst, W, a_src, a_dst, b):
    h, asr, adr, mv = _prep(x, W, a_src.reshape(1, D), a_dst.reshape(1, D))
    wts, denp = _wpass(src, dst, asr, adr, mv)
    accp = _rpass(h, src, dst, wts)
    return _fin(accp, denp, b.reshape(1, D))


def kernel(x, edge_index, W1, a_src1, a_dst1, b1, W2, a_src2, a_dst2, b2):
    loop = jnp.arange(N, dtype=jnp.int32)
    pad = jnp.zeros((EPAD - EXT,), jnp.int32)
    src = jnp.concatenate([edge_index[0], loop, pad])
    dst = jnp.concatenate([edge_index[1], loop, pad])
    z = _gat_layer(x, src, dst, W1, a_src1, a_dst1, b1)
    xbar = _gat_layer(z, src, dst, W2, a_src2, a_dst2, b2)
    return xbar, z


# speedup vs baseline: 1.3161x; 1.3161x over previous
import dataclasses
import functools

import jax
import jax.numpy as jnp
from jax import lax
from jax.experimental import pallas as pl
from jax.experimental.pallas import tpu as pltpu
from jax.experimental.pallas import tpu_sc as plsc

N = 10000
E = 320000
D = 128

NC = 2
NS = 16
L = 16
NW = NC * NS

EXT = E + N
EPT = 10368
EPAD = EPT * NW
CW = 576
NCHW = EPT // CW
C = 96
NCH = EPT // C
SB = 624
ET = N - NS * SB


def _prep_body(x_ref, w_ref, asrc_ref, adst_ref, h_ref, as_ref, ad_ref, mv_ref):
    x = x_ref[...]
    w = w_ref[...]
    h = lax.dot_general(x, w, (((1,), (1,)), ((), ())),
                        preferred_element_type=jnp.float32)
    h_ref[...] = h
    asr = lax.dot_general(asrc_ref[...], h, (((1,), (1,)), ((), ())),
                          preferred_element_type=jnp.float32)
    adr = lax.dot_general(adst_ref[...], h, (((1,), (1,)), ((), ())),
                          preferred_element_type=jnp.float32)
    as_ref[...] = asr
    ad_ref[...] = adr
    m = jnp.max(asr) + jnp.max(adr)
    m = jnp.maximum(m, 0.2 * m)
    mv_ref[...] = jnp.full((8, 128), m, jnp.float32)


_prep = pl.pallas_call(
    _prep_body,
    out_shape=[
        jax.ShapeDtypeStruct((N, D), jnp.float32),
        jax.ShapeDtypeStruct((1, N), jnp.float32),
        jax.ShapeDtypeStruct((1, N), jnp.float32),
        jax.ShapeDtypeStruct((8, 128), jnp.float32),
    ],
)


def _fin_body(p_ref, den_ref, b_ref, o_ref):
    num = p_ref[0] + p_ref[1]
    ones = jnp.ones((NW, 1), jnp.float32)
    dcol = lax.dot_general(den_ref[...], ones, (((0,), (0,)), ((), ())),
                           preferred_element_type=jnp.float32)
    o = num / (dcol + 1e-16) + b_ref[...]
    o_ref[...] = jnp.where(o > 0, o, jnp.exp(jnp.minimum(o, 0.0)) - 1.0)


_fin = pl.pallas_call(
    _fin_body,
    out_shape=jax.ShapeDtypeStruct((N, D), jnp.float32),
)


_sc_params = pltpu.CompilerParams()
if "needs_layout_passes" in pltpu.CompilerParams.__dataclass_fields__:
    _sc_params = dataclasses.replace(_sc_params, needs_layout_passes=False)

_MESH = plsc.VectorSubcoreMesh(core_axis_name="c", subcore_axis_name="s")


def _wpass_body(src_hbm, dst_hbm, as_hbm, ad_hbm, mv_hbm,
                w_hbm, denp_hbm,
                as_v, ad_v, mv_v, src0_v, src1_v, dst0_v, dst1_v,
                w0_v, w1_v, den_v, si0, si1, so0, so1):
    cid = lax.axis_index("c")
    sid = lax.axis_index("s")
    wid = cid * NS + sid
    srcs, dsts = [src0_v, src1_v], [dst0_v, dst1_v]
    wouts, si, so = [w0_v, w1_v], [si0, si1], [so0, so1]

    pltpu.sync_copy(as_hbm.at[0], as_v)
    pltpu.sync_copy(ad_hbm.at[0], ad_v)
    pltpu.sync_copy(mv_hbm.at[0, pl.ds(0, L)], mv_v)

    zero16 = jnp.zeros((L,), jnp.float32)

    @pl.loop(0, N // L)
    def _zden(r):
        den_v[pl.ds(r * L, L)] = zero16

    mvec = mv_v[...]

    def _issue_idx(c, p):
        base = wid * EPT + c * CW
        pltpu.async_copy(src_hbm.at[pl.ds(base, CW)], srcs[p], si[p])
        pltpu.async_copy(dst_hbm.at[pl.ds(base, CW)], dsts[p], si[p])

    def _wait_idx(p):
        pltpu.make_async_copy(src_hbm.at[pl.ds(0, CW)], srcs[p], si[p]).wait()
        pltpu.make_async_copy(dst_hbm.at[pl.ds(0, CW)], dsts[p], si[p]).wait()

    def _body(c, p, pf_idx, wait_out):
        _wait_idx(p)
        if wait_out:
            pltpu.make_async_copy(w_hbm.at[pl.ds(0, CW)], wouts[p],
                                  so[p]).wait()
        for g in range(CW // L):
            s16 = srcs[p][pl.ds(g * L, L)]
            d16 = dsts[p][pl.ds(g * L, L)]
            av = plsc.load_gather(as_v, [s16])
            dv = plsc.load_gather(ad_v, [d16])
            t = av + dv
            alpha = jnp.maximum(t, 0.2 * t)
            wv = jnp.exp(alpha - mvec)
            eid = wid * EPT + c * CW + g * L + lax.iota(jnp.int32, L)
            wv = jnp.where(eid < EXT, wv, 0.0)
            wouts[p][pl.ds(g * L, L)] = wv
            plsc.addupdate_scatter(den_v, [d16], wv)
        base = wid * EPT + c * CW
        pltpu.async_copy(wouts[p], w_hbm.at[pl.ds(base, CW)], so[p])
        if pf_idx:
            _issue_idx(c + 2, p)

    _issue_idx(0, 0)
    _issue_idx(1, 1)
    _body(0, 0, pf_idx=True, wait_out=False)
    _body(1, 1, pf_idx=True, wait_out=False)

    @pl.loop(0, (NCHW - 4) // 2)
    def _main(j):
        for b in range(2):
            _body(2 + 2 * j + b, b, pf_idx=True, wait_out=True)

    _body(NCHW - 2, 0, pf_idx=False, wait_out=True)
    _body(NCHW - 1, 1, pf_idx=False, wait_out=True)
    pltpu.make_async_copy(w_hbm.at[pl.ds(0, CW)], wouts[0], so[0]).wait()
    pltpu.make_async_copy(w_hbm.at[pl.ds(0, CW)], wouts[1], so[1]).wait()

    pltpu.sync_copy(den_v, denp_hbm.at[wid])


_wpass = pl.kernel(
    _wpass_body,
    out_type=[
        jax.ShapeDtypeStruct((EPAD,), jnp.float32),
        jax.ShapeDtypeStruct((NW, N), jnp.float32),
    ],
    mesh=_MESH,
    compiler_params=_sc_params,
    scratch_types=[
        pltpu.VMEM((N,), jnp.float32),
        pltpu.VMEM((N,), jnp.float32),
        pltpu.VMEM((L,), jnp.float32),
        pltpu.VMEM((CW,), jnp.int32),
        pltpu.VMEM((CW,), jnp.int32),
        pltpu.VMEM((CW,), jnp.int32),
        pltpu.VMEM((CW,), jnp.int32),
        pltpu.VMEM((CW,), jnp.float32),
        pltpu.VMEM((CW,), jnp.float32),
        pltpu.VMEM((N,), jnp.float32),
        pltpu.SemaphoreType.DMA,
        pltpu.SemaphoreType.DMA,
        pltpu.SemaphoreType.DMA,
        pltpu.SemaphoreType.DMA,
    ],
)


def _rpass_body(h_hbm, src_hbm, dst_hbm, w_hbm,
                accp_hbm,
                src_v, dst_v, w_v, rows0_v, rows1_v, rows2_v,
                acc_sh, si0, si1, si2, si3, si4, si5,
                sg0, sg1, sg2, sc0, sc1, sc2):
    cid = lax.axis_index("c")
    sid = lax.axis_index("s")
    wid = cid * NS + sid
    rows = [rows0_v, rows1_v, rows2_v]
    si = [si0, si1, si2, si3, si4, si5]
    sg, sc = [sg0, sg1, sg2], [sc0, sc1, sc2]

    zero16 = jnp.zeros((L,), jnp.float32)

    @pl.loop(0, C)
    def _zrow(r):
        for k in range(D // L):
            rows0_v[r, pl.ds(k * L, L)] = zero16

    r0 = sid * SB

    @pl.loop(0, SB // C)
    def _zacc(j):
        pltpu.sync_copy(rows0_v, acc_sh.at[pl.ds(r0 + j * C, C)])

    zrem = SB % C
    if zrem:
        pltpu.sync_copy(rows0_v.at[pl.ds(0, zrem)],
                        acc_sh.at[pl.ds(r0 + (SB // C) * C, zrem)])

    @pl.when(sid == NS - 1)
    def _ztail():
        pltpu.sync_copy(rows0_v.at[pl.ds(0, ET)], acc_sh.at[pl.ds(NS * SB, ET)])

    plsc.subcore_barrier()

    def _issue_idx(c, p):
        base = wid * EPT + c * C
        pltpu.async_copy(src_hbm.at[pl.ds(base, C)], src_v.at[p], si[p])
        pltpu.async_copy(dst_hbm.at[pl.ds(base, C)], dst_v.at[p], si[p])
        pltpu.async_copy(w_hbm.at[pl.ds(base, C)], w_v.at[p], si[p])

    def _wait_idx(p):
        pltpu.make_async_copy(src_hbm.at[pl.ds(0, C)], src_v.at[p],
                              si[p]).wait()
        pltpu.make_async_copy(dst_hbm.at[pl.ds(0, C)], dst_v.at[p],
                              si[p]).wait()
        pltpu.make_async_copy(w_hbm.at[pl.ds(0, C)], w_v.at[p], si[p]).wait()

    H2 = C // 2

    def _issue_gat(p6, p3):
        pltpu.async_copy(h_hbm.at[src_v.at[p6, pl.ds(0, H2)]],
                         rows[p3].at[pl.ds(0, H2)], sg[p3])
        pltpu.async_copy(h_hbm.at[src_v.at[p6, pl.ds(H2, H2)]],
                         rows[p3].at[pl.ds(H2, H2)], sg[p3])

    def _wait_gat(p6, p3):
        pltpu.make_async_copy(h_hbm.at[src_v.at[p6, pl.ds(0, H2)]],
                              rows[p3].at[pl.ds(0, H2)], sg[p3]).wait()
        pltpu.make_async_copy(h_hbm.at[src_v.at[p6, pl.ds(H2, H2)]],
                              rows[p3].at[pl.ds(H2, H2)], sg[p3]).wait()

    def _wait_sc(p6, p3):
        pltpu.make_async_copy(rows[p3], acc_sh.at[dst_v.at[p6]], sc[p3]).wait()

    def _body(c, p6, p3, wait_sc, pf_gat, pf_idx):
        _wait_gat(p6, p3)
        if wait_sc:
            _wait_sc((p6 + 4) % 6, (p3 + 1) % 3)
        if pf_gat:
            _wait_idx((p6 + 1) % 6)
            _issue_gat((p6 + 1) % 6, (p3 + 1) % 3)

        @pl.loop(0, C)
        def _scale(e):
            s = plsc.load_gather(w_v.at[p6], [jnp.full((L,), e, jnp.int32)])
            for k in range(D // L):
                rows[p3][e, pl.ds(k * L, L)] = rows[p3][e, pl.ds(k * L, L)] * s

        pltpu.async_copy(rows[p3], acc_sh.at[dst_v.at[p6]], sc[p3], add=True)
        if pf_idx:
            _issue_idx(c + 3, (p6 + 3) % 6)

    _issue_idx(0, 0)
    _issue_idx(1, 1)
    _issue_idx(2, 2)
    _wait_idx(0)
    _issue_gat(0, 0)
    _body(0, 0, 0, wait_sc=False, pf_gat=True, pf_idx=True)
    _body(1, 1, 1, wait_sc=False, pf_gat=True, pf_idx=True)

    @pl.loop(0, (NCH - 6) // 6)
    def _main(j):
        for u in range(6):
            c = 2 + 6 * j + u
            _body(c, (2 + u) % 6, (2 + u) % 3,
                  wait_sc=True, pf_gat=True, pf_idx=True)

    for u in range(4):
        c = NCH - 4 + u
        _body(c, c % 6, c % 3, wait_sc=True,
              pf_gat=(u < 3), pf_idx=(u == 0))

    _wait_sc((NCH - 2) % 6, (NCH - 2) % 3)
    _wait_sc((NCH - 1) % 6, (NCH - 1) % 3)

    plsc.subcore_barrier()
    pltpu.sync_copy(acc_sh.at[pl.ds(r0, SB)], accp_hbm.at[cid, pl.ds(r0, SB)])

    @pl.when(sid == NS - 1)
    def _dtail():
        t0 = NS * SB
        pltpu.sync_copy(acc_sh.at[pl.ds(t0, ET)], accp_hbm.at[cid, pl.ds(t0, ET)])


_rpass = pl.kernel(
    _rpass_body,
    out_type=jax.ShapeDtypeStruct((NC, N, D), jnp.float32),
    mesh=_MESH,
    compiler_params=_sc_params,
    scratch_types=[
        pltpu.VMEM((6, C), jnp.int32),
        pltpu.VMEM((6, C), jnp.int32),
        pltpu.VMEM((6, C), jnp.float32),
        pltpu.VMEM((C, D), jnp.float32),
        pltpu.VMEM((C, D), jnp.float32),
        pltpu.VMEM((C, D), jnp.float32),
        pltpu.VMEM_SHARED((N, D), jnp.float32),
        pltpu.SemaphoreType.DMA,
        pltpu.SemaphoreType.DMA,
        pltpu.SemaphoreType.DMA,
        pltpu.SemaphoreType.DMA,
        pltpu.SemaphoreType.DMA,
        pltpu.SemaphoreType.DMA,
        pltpu.SemaphoreType.DMA,
        pltpu.SemaphoreType.DMA,
        pltpu.SemaphoreType.DMA,
        pltpu.SemaphoreType.DMA,
        pltpu.SemaphoreType.DMA,
        pltpu.SemaphoreType.DMA,
    ],
)


def _gat_layer(x, src, dst, W, a_src, a_dst, b):
    h, asr, adr, mv = _prep(x, W, a_src.reshape(1, D), a_dst.reshape(1, D))
    wts, denp = _wpass(src, dst, asr, adr, mv)
    accp = _rpass(h, src, dst, wts)
    return _fin(accp, denp, b.reshape(1, D))


def kernel(x, edge_index, W1, a_src1, a_dst1, b1, W2, a_src2, a_dst2, b2):
    loop = jnp.arange(N, dtype=jnp.int32)
    pad = jnp.zeros((EPAD - EXT,), jnp.int32)
    src = jnp.concatenate([edge_index[0], loop, pad])
    dst = jnp.concatenate([edge_index[1], loop, pad])
    z = _gat_layer(x, src, dst, W1, a_src1, a_dst1, b1)
    xbar = _gat_layer(z, src, dst, W2, a_src2, a_dst2, b2)
    return xbar, z


# speedup vs baseline: 1.3502x; 1.0259x over previous
import dataclasses
import functools

import jax
import jax.numpy as jnp
from jax import lax
from jax.experimental import pallas as pl
from jax.experimental.pallas import tpu as pltpu
from jax.experimental.pallas import tpu_sc as plsc

N = 10000
E = 320000
D = 128

NC = 2
NS = 16
L = 16
NW = NC * NS

EXT = E + N
EPT = 10368
EPAD = EPT * NW
CW = 576
NCHW = EPT // CW
C = 96
NCH = EPT // C
SB = 624
ET = N - NS * SB


def _prep_body(x_ref, w_ref, asrc_ref, adst_ref, h_ref, as_ref, ad_ref, mv_ref):
    x = x_ref[...]
    w = w_ref[...]
    h = lax.dot_general(x, w, (((1,), (1,)), ((), ())),
                        preferred_element_type=jnp.float32)
    h_ref[...] = h
    asr = lax.dot_general(asrc_ref[...], h, (((1,), (1,)), ((), ())),
                          preferred_element_type=jnp.float32)
    adr = lax.dot_general(adst_ref[...], h, (((1,), (1,)), ((), ())),
                          preferred_element_type=jnp.float32)
    as_ref[...] = asr
    ad_ref[...] = adr
    m = jnp.max(asr) + jnp.max(adr)
    m = jnp.maximum(m, 0.2 * m)
    mv_ref[...] = jnp.full((8, 128), m, jnp.float32)


_prep = pl.pallas_call(
    _prep_body,
    out_shape=[
        jax.ShapeDtypeStruct((N, D), jnp.float32),
        jax.ShapeDtypeStruct((1, N), jnp.float32),
        jax.ShapeDtypeStruct((1, N), jnp.float32),
        jax.ShapeDtypeStruct((8, 128), jnp.float32),
    ],
)


def _fin_body(p_ref, den_ref, b_ref, o_ref):
    num = p_ref[0] + p_ref[1]
    ones = jnp.ones((NW, 1), jnp.float32)
    dcol = lax.dot_general(den_ref[...], ones, (((0,), (0,)), ((), ())),
                           preferred_element_type=jnp.float32)
    o = num / (dcol + 1e-16) + b_ref[...]
    o_ref[...] = jnp.where(o > 0, o, jnp.exp(jnp.minimum(o, 0.0)) - 1.0)


_fin = pl.pallas_call(
    _fin_body,
    out_shape=jax.ShapeDtypeStruct((N, D), jnp.float32),
)


_sc_params = pltpu.CompilerParams()
if "needs_layout_passes" in pltpu.CompilerParams.__dataclass_fields__:
    _sc_params = dataclasses.replace(_sc_params, needs_layout_passes=False)

_MESH = plsc.VectorSubcoreMesh(core_axis_name="c", subcore_axis_name="s")


def _wpass_body(src_hbm, dst_hbm, as_hbm, ad_hbm, mv_hbm,
                w_hbm, denp_hbm,
                as_v, ad_v, mv_v, src0_v, src1_v, dst0_v, dst1_v,
                w0_v, w1_v, den_v, si0, si1, so0, so1):
    cid = lax.axis_index("c")
    sid = lax.axis_index("s")
    wid = cid * NS + sid
    srcs, dsts = [src0_v, src1_v], [dst0_v, dst1_v]
    wouts, si, so = [w0_v, w1_v], [si0, si1], [so0, so1]

    pltpu.sync_copy(as_hbm.at[0], as_v)
    pltpu.sync_copy(ad_hbm.at[0], ad_v)
    pltpu.sync_copy(mv_hbm.at[0, pl.ds(0, L)], mv_v)

    zero16 = jnp.zeros((L,), jnp.float32)

    @pl.loop(0, N // L)
    def _zden(r):
        den_v[pl.ds(r * L, L)] = zero16

    mvec = mv_v[...]

    def _issue_idx(c, p):
        base = wid * EPT + c * CW
        pltpu.async_copy(src_hbm.at[pl.ds(base, CW)], srcs[p], si[p])
        pltpu.async_copy(dst_hbm.at[pl.ds(base, CW)], dsts[p], si[p])

    def _wait_idx(p):
        pltpu.make_async_copy(src_hbm.at[pl.ds(0, CW)], srcs[p], si[p]).wait()
        pltpu.make_async_copy(dst_hbm.at[pl.ds(0, CW)], dsts[p], si[p]).wait()

    def _body(c, p, pf_idx, wait_out):
        _wait_idx(p)
        if wait_out:
            pltpu.make_async_copy(w_hbm.at[pl.ds(0, CW)], wouts[p],
                                  so[p]).wait()
        for g in range(CW // L):
            s16 = srcs[p][pl.ds(g * L, L)]
            d16 = dsts[p][pl.ds(g * L, L)]
            av = plsc.load_gather(as_v, [s16])
            dv = plsc.load_gather(ad_v, [d16])
            t = av + dv
            alpha = jnp.maximum(t, 0.2 * t)
            wv = jnp.exp(alpha - mvec)
            eid = wid * EPT + c * CW + g * L + lax.iota(jnp.int32, L)
            wv = jnp.where(eid < EXT, wv, 0.0)
            wouts[p][pl.ds(g * L, L)] = wv
            plsc.addupdate_scatter(den_v, [d16], wv)
        base = wid * EPT + c * CW
        pltpu.async_copy(wouts[p], w_hbm.at[pl.ds(base, CW)], so[p])
        if pf_idx:
            _issue_idx(c + 2, p)

    _issue_idx(0, 0)
    _issue_idx(1, 1)
    _body(0, 0, pf_idx=True, wait_out=False)
    _body(1, 1, pf_idx=True, wait_out=False)

    @pl.loop(0, (NCHW - 4) // 2)
    def _main(j):
        for b in range(2):
            _body(2 + 2 * j + b, b, pf_idx=True, wait_out=True)

    _body(NCHW - 2, 0, pf_idx=False, wait_out=True)
    _body(NCHW - 1, 1, pf_idx=False, wait_out=True)
    pltpu.make_async_copy(w_hbm.at[pl.ds(0, CW)], wouts[0], so[0]).wait()
    pltpu.make_async_copy(w_hbm.at[pl.ds(0, CW)], wouts[1], so[1]).wait()

    pltpu.sync_copy(den_v, denp_hbm.at[wid])


_wpass = pl.kernel(
    _wpass_body,
    out_type=[
        jax.ShapeDtypeStruct((EPAD,), jnp.float32),
        jax.ShapeDtypeStruct((NW, N), jnp.float32),
    ],
    mesh=_MESH,
    compiler_params=_sc_params,
    scratch_types=[
        pltpu.VMEM((N,), jnp.float32),
        pltpu.VMEM((N,), jnp.float32),
        pltpu.VMEM((L,), jnp.float32),
        pltpu.VMEM((CW,), jnp.int32),
        pltpu.VMEM((CW,), jnp.int32),
        pltpu.VMEM((CW,), jnp.int32),
        pltpu.VMEM((CW,), jnp.int32),
        pltpu.VMEM((CW,), jnp.float32),
        pltpu.VMEM((CW,), jnp.float32),
        pltpu.VMEM((N,), jnp.float32),
        pltpu.SemaphoreType.DMA,
        pltpu.SemaphoreType.DMA,
        pltpu.SemaphoreType.DMA,
        pltpu.SemaphoreType.DMA,
    ],
)


def _rpass_body(h_hbm, src_hbm, dst_hbm, w_hbm,
                accp_hbm,
                src_v, dst_v, w_v, rf0_v, rf1_v, rf2_v,
                acc_sh, si0, si1, si2, si3, si4, si5,
                sg0, sg1, sg2, sc0, sc1, sc2):
    cid = lax.axis_index("c")
    sid = lax.axis_index("s")
    wid = cid * NS + sid
    rf = [rf0_v, rf1_v, rf2_v]
    si = [si0, si1, si2, si3, si4, si5]
    sg, sc = [sg0, sg1, sg2], [sc0, sc1, sc2]

    zero16 = jnp.zeros((L,), jnp.float32)

    @pl.loop(0, C)
    def _zrow(r):
        for k in range(D // L):
            rf0_v[r, pl.ds(k * L, L)] = zero16

    r0 = sid * SB

    @pl.loop(0, SB // C)
    def _zacc(j):
        pltpu.sync_copy(rf0_v, acc_sh.at[pl.ds(r0 + j * C, C)])

    zrem = SB % C
    if zrem:
        pltpu.sync_copy(rf0_v.at[pl.ds(0, zrem)],
                        acc_sh.at[pl.ds(r0 + (SB // C) * C, zrem)])

    @pl.when(sid == NS - 1)
    def _ztail():
        pltpu.sync_copy(rf0_v.at[pl.ds(0, ET)], acc_sh.at[pl.ds(NS * SB, ET)])

    plsc.subcore_barrier()

    def _issue_idx(c, p):
        base = wid * EPT + c * C
        pltpu.async_copy(src_hbm.at[pl.ds(base, C)], src_v.at[p], si[p])
        pltpu.async_copy(dst_hbm.at[pl.ds(base, C)], dst_v.at[p], si[p])
        pltpu.async_copy(w_hbm.at[pl.ds(base, C)], w_v.at[p], si[p])

    def _wait_idx(p):
        pltpu.make_async_copy(src_hbm.at[pl.ds(0, C)], src_v.at[p],
                              si[p]).wait()
        pltpu.make_async_copy(dst_hbm.at[pl.ds(0, C)], dst_v.at[p],
                              si[p]).wait()
        pltpu.make_async_copy(w_hbm.at[pl.ds(0, C)], w_v.at[p], si[p]).wait()

    def _issue_gat(p6, p3):
        pltpu.async_copy(h_hbm.at[src_v.at[p6]], rf[p3], sg[p3])

    def _wait_gat(p6, p3):
        pltpu.make_async_copy(h_hbm.at[src_v.at[p6]], rf[p3], sg[p3]).wait()

    def _wait_sc(p6, p3):
        pltpu.make_async_copy(rf[p3], acc_sh.at[dst_v.at[p6]], sc[p3]).wait()

    def _body(c, p6, p3, wait_sc, pf_gat, pf_idx):
        _wait_gat(p6, p3)
        if wait_sc:
            _wait_sc((p6 + 4) % 6, (p3 + 1) % 3)
        if pf_gat:
            _wait_idx((p6 + 1) % 6)
            _issue_gat((p6 + 1) % 6, (p3 + 1) % 3)

        @pl.loop(0, C)
        def _scale(e):
            s = plsc.load_gather(w_v.at[p6], [jnp.full((L,), e, jnp.int32)])
            for k in range(D // L):
                rf[p3][e, pl.ds(k * L, L)] = rf[p3][e, pl.ds(k * L, L)] * s

        pltpu.async_copy(rf[p3], acc_sh.at[dst_v.at[p6]], sc[p3], add=True)
        if pf_idx:
            _issue_idx(c + 3, (p6 + 3) % 6)

    _issue_idx(0, 0)
    _issue_idx(1, 1)
    _issue_idx(2, 2)
    _wait_idx(0)
    _issue_gat(0, 0)
    _body(0, 0, 0, wait_sc=False, pf_gat=True, pf_idx=True)
    _body(1, 1, 1, wait_sc=False, pf_gat=True, pf_idx=True)

    @pl.loop(0, (NCH - 6) // 6)
    def _main(j):
        for u in range(6):
            c = 2 + 6 * j + u
            _body(c, (2 + u) % 6, (2 + u) % 3,
                  wait_sc=True, pf_gat=True, pf_idx=True)

    for u in range(4):
        c = NCH - 4 + u
        _body(c, c % 6, c % 3, wait_sc=True,
              pf_gat=(u < 3), pf_idx=(u == 0))

    _wait_sc((NCH - 2) % 6, (NCH - 2) % 3)
    _wait_sc((NCH - 1) % 6, (NCH - 1) % 3)

    plsc.subcore_barrier()
    pltpu.sync_copy(acc_sh.at[pl.ds(r0, SB)], accp_hbm.at[cid, pl.ds(r0, SB)])

    @pl.when(sid == NS - 1)
    def _dtail():
        t0 = NS * SB
        pltpu.sync_copy(acc_sh.at[pl.ds(t0, ET)], accp_hbm.at[cid, pl.ds(t0, ET)])


_rpass = pl.kernel(
    _rpass_body,
    out_type=jax.ShapeDtypeStruct((NC, N, D), jnp.float32),
    mesh=_MESH,
    compiler_params=_sc_params,
    scratch_types=[
        pltpu.VMEM((6, C), jnp.int32),
        pltpu.VMEM((6, C), jnp.int32),
        pltpu.VMEM((6, C), jnp.float32),
        pltpu.VMEM((C, D), jnp.float32),
        pltpu.VMEM((C, D), jnp.float32),
        pltpu.VMEM((C, D), jnp.float32),
        pltpu.VMEM_SHARED((N, D), jnp.float32),
        pltpu.SemaphoreType.DMA,
        pltpu.SemaphoreType.DMA,
        pltpu.SemaphoreType.DMA,
        pltpu.SemaphoreType.DMA,
        pltpu.SemaphoreType.DMA,
        pltpu.SemaphoreType.DMA,
        pltpu.SemaphoreType.DMA,
        pltpu.SemaphoreType.DMA,
        pltpu.SemaphoreType.DMA,
        pltpu.SemaphoreType.DMA,
        pltpu.SemaphoreType.DMA,
        pltpu.SemaphoreType.DMA,
    ],
)


def _gat_layer(x, src, dst, W, a_src, a_dst, b):
    h, asr, adr, mv = _prep(x, W, a_src.reshape(1, D), a_dst.reshape(1, D))
    wts, denp = _wpass(src, dst, asr, adr, mv)
    accp = _rpass(h, src, dst, wts)
    return _fin(accp, denp, b.reshape(1, D))


def kernel(x, edge_index, W1, a_src1, a_dst1, b1, W2, a_src2, a_dst2, b2):
    loop = jnp.arange(N, dtype=jnp.int32)
    pad = jnp.zeros((EPAD - EXT,), jnp.int32)
    src = jnp.concatenate([edge_index[0], loop, pad])
    dst = jnp.concatenate([edge_index[1], loop, pad])
    z = _gat_layer(x, src, dst, W1, a_src1, a_dst1, b1)
    xbar = _gat_layer(z, src, dst, W2, a_src2, a_dst2, b2)
    return xbar, z


# speedup vs baseline: 1.7356x; 1.2855x over previous
import dataclasses
import functools

import jax
import jax.numpy as jnp
from jax import lax
from jax.experimental import pallas as pl
from jax.experimental.pallas import tpu as pltpu
from jax.experimental.pallas import tpu_sc as plsc

N = 10000
E = 320000
D = 128

NC = 2
NS = 16
L = 16
NW = NC * NS

EXT = E + N
EPT = 10368
EPAD = EPT * NW
CW = 576
NCHW = EPT // CW
C = 96
NCH = EPT // C
SB = 624
ET = N - NS * SB


def _prep_body(x_ref, w_ref, asrc_ref, adst_ref, h_ref, as_ref, ad_ref, mv_ref):
    x = x_ref[...]
    w = w_ref[...]
    h = lax.dot_general(x, w, (((1,), (1,)), ((), ())),
                        preferred_element_type=jnp.float32)
    h_ref[...] = h
    asr = lax.dot_general(asrc_ref[...], h, (((1,), (1,)), ((), ())),
                          preferred_element_type=jnp.float32)
    adr = lax.dot_general(adst_ref[...], h, (((1,), (1,)), ((), ())),
                          preferred_element_type=jnp.float32)
    as_ref[...] = asr
    ad_ref[...] = adr
    m = jnp.max(asr) + jnp.max(adr)
    m = jnp.maximum(m, 0.2 * m)
    mv_ref[...] = jnp.full((8, 128), m, jnp.float32)


_prep = pl.pallas_call(
    _prep_body,
    out_shape=[
        jax.ShapeDtypeStruct((N, D), jnp.float32),
        jax.ShapeDtypeStruct((1, N), jnp.float32),
        jax.ShapeDtypeStruct((1, N), jnp.float32),
        jax.ShapeDtypeStruct((8, 128), jnp.float32),
    ],
)


def _fin_body(p_ref, den_ref, b_ref, o_ref):
    num = p_ref[0] + p_ref[1]
    ones = jnp.ones((NW, 1), jnp.float32)
    dcol = lax.dot_general(den_ref[...], ones, (((0,), (0,)), ((), ())),
                           preferred_element_type=jnp.float32)
    o = num / (dcol + 1e-16) + b_ref[...]
    o_ref[...] = jnp.where(o > 0, o, jnp.exp(jnp.minimum(o, 0.0)) - 1.0)


_fin = pl.pallas_call(
    _fin_body,
    out_shape=jax.ShapeDtypeStruct((N, D), jnp.float32),
)


_sc_params = pltpu.CompilerParams()
if "needs_layout_passes" in pltpu.CompilerParams.__dataclass_fields__:
    _sc_params = dataclasses.replace(_sc_params, needs_layout_passes=False)

_MESH = plsc.VectorSubcoreMesh(core_axis_name="c", subcore_axis_name="s")


def _wpass_body(src_hbm, dst_hbm, as_hbm, ad_hbm, mv_hbm,
                w_hbm, denp_hbm,
                as_v, ad_v, mv_v, src0_v, src1_v, dst0_v, dst1_v,
                w0_v, w1_v, den_v, si0, si1, so0, so1):
    cid = lax.axis_index("c")
    sid = lax.axis_index("s")
    wid = cid * NS + sid
    srcs, dsts = [src0_v, src1_v], [dst0_v, dst1_v]
    wouts, si, so = [w0_v, w1_v], [si0, si1], [so0, so1]

    pltpu.sync_copy(as_hbm.at[0], as_v)
    pltpu.sync_copy(ad_hbm.at[0], ad_v)
    pltpu.sync_copy(mv_hbm.at[0, pl.ds(0, L)], mv_v)

    zero16 = jnp.zeros((L,), jnp.float32)

    @pl.loop(0, N // L)
    def _zden(r):
        den_v[pl.ds(r * L, L)] = zero16

    mvec = mv_v[...]

    def _issue_idx(c, p):
        base = wid * EPT + c * CW
        pltpu.async_copy(src_hbm.at[pl.ds(base, CW)], srcs[p], si[p])
        pltpu.async_copy(dst_hbm.at[pl.ds(base, CW)], dsts[p], si[p])

    def _wait_idx(p):
        pltpu.make_async_copy(src_hbm.at[pl.ds(0, CW)], srcs[p], si[p]).wait()
        pltpu.make_async_copy(dst_hbm.at[pl.ds(0, CW)], dsts[p], si[p]).wait()

    def _body(c, p, pf_idx, wait_out):
        _wait_idx(p)
        if wait_out:
            pltpu.make_async_copy(w_hbm.at[pl.ds(0, CW)], wouts[p],
                                  so[p]).wait()
        for g in range(CW // L):
            s16 = srcs[p][pl.ds(g * L, L)]
            d16 = dsts[p][pl.ds(g * L, L)]
            av = plsc.load_gather(as_v, [s16])
            dv = plsc.load_gather(ad_v, [d16])
            t = av + dv
            alpha = jnp.maximum(t, 0.2 * t)
            wv = jnp.exp(alpha - mvec)
            eid = wid * EPT + c * CW + g * L + lax.iota(jnp.int32, L)
            wv = jnp.where(eid < EXT, wv, 0.0)
            wouts[p][pl.ds(g * L, L)] = wv
            plsc.addupdate_scatter(den_v, [d16], wv)
        base = wid * EPT + c * CW
        pltpu.async_copy(wouts[p], w_hbm.at[pl.ds(base, CW)], so[p])
        if pf_idx:
            _issue_idx(c + 2, p)

    _issue_idx(0, 0)
    _issue_idx(1, 1)
    _body(0, 0, pf_idx=True, wait_out=False)
    _body(1, 1, pf_idx=True, wait_out=False)

    @pl.loop(0, (NCHW - 4) // 2)
    def _main(j):
        for b in range(2):
            _body(2 + 2 * j + b, b, pf_idx=True, wait_out=True)

    _body(NCHW - 2, 0, pf_idx=False, wait_out=True)
    _body(NCHW - 1, 1, pf_idx=False, wait_out=True)
    pltpu.make_async_copy(w_hbm.at[pl.ds(0, CW)], wouts[0], so[0]).wait()
    pltpu.make_async_copy(w_hbm.at[pl.ds(0, CW)], wouts[1], so[1]).wait()

    pltpu.sync_copy(den_v, denp_hbm.at[wid])


_wpass = pl.kernel(
    _wpass_body,
    out_type=[
        jax.ShapeDtypeStruct((EPAD,), jnp.float32),
        jax.ShapeDtypeStruct((NW, N), jnp.float32),
    ],
    mesh=_MESH,
    compiler_params=_sc_params,
    scratch_types=[
        pltpu.VMEM((N,), jnp.float32),
        pltpu.VMEM((N,), jnp.float32),
        pltpu.VMEM((L,), jnp.float32),
        pltpu.VMEM((CW,), jnp.int32),
        pltpu.VMEM((CW,), jnp.int32),
        pltpu.VMEM((CW,), jnp.int32),
        pltpu.VMEM((CW,), jnp.int32),
        pltpu.VMEM((CW,), jnp.float32),
        pltpu.VMEM((CW,), jnp.float32),
        pltpu.VMEM((N,), jnp.float32),
        pltpu.SemaphoreType.DMA,
        pltpu.SemaphoreType.DMA,
        pltpu.SemaphoreType.DMA,
        pltpu.SemaphoreType.DMA,
    ],
)


def _rpass_body(h_hbm, src_hbm, dst_hbm, w_hbm,
                accp_hbm,
                src_v, dst_v, w_v, rf0_v, rf1_v, rf2_v,
                acc_sh, si0, si1, si2, si3, si4, si5,
                sg0, sg1, sg2, sc0, sc1, sc2):
    cid = lax.axis_index("c")
    sid = lax.axis_index("s")
    wid = cid * NS + sid
    rf = [rf0_v, rf1_v, rf2_v]
    si = [si0, si1, si2, si3, si4, si5]
    sg, sc = [sg0, sg1, sg2], [sc0, sc1, sc2]

    zero16 = jnp.zeros((L,), jnp.float32)

    @pl.loop(0, C)
    def _zrow(r):
        for k in range(D // L):
            rf0_v[r, pl.ds(k * L, L)] = zero16

    r0 = sid * SB

    @pl.loop(0, SB // C)
    def _zacc(j):
        pltpu.sync_copy(rf0_v, acc_sh.at[pl.ds(r0 + j * C, C)])

    zrem = SB % C
    if zrem:
        pltpu.sync_copy(rf0_v.at[pl.ds(0, zrem)],
                        acc_sh.at[pl.ds(r0 + (SB // C) * C, zrem)])

    @pl.when(sid == NS - 1)
    def _ztail():
        pltpu.sync_copy(rf0_v.at[pl.ds(0, ET)], acc_sh.at[pl.ds(NS * SB, ET)])

    plsc.subcore_barrier()

    def _issue_idx(c, p):
        base = wid * EPT + c * C
        pltpu.async_copy(src_hbm.at[pl.ds(base, C)], src_v.at[p], si[p])
        pltpu.async_copy(dst_hbm.at[pl.ds(base, C)], dst_v.at[p], si[p])
        pltpu.async_copy(w_hbm.at[pl.ds(base, C)], w_v.at[p], si[p])

    def _wait_idx(p):
        pltpu.make_async_copy(src_hbm.at[pl.ds(0, C)], src_v.at[p],
                              si[p]).wait()
        pltpu.make_async_copy(dst_hbm.at[pl.ds(0, C)], dst_v.at[p],
                              si[p]).wait()
        pltpu.make_async_copy(w_hbm.at[pl.ds(0, C)], w_v.at[p], si[p]).wait()

    def _issue_gat(p6, p3):
        pltpu.async_copy(h_hbm.at[src_v.at[p6]], rf[p3], sg[p3])

    def _wait_gat(p6, p3):
        pltpu.make_async_copy(h_hbm.at[src_v.at[p6]], rf[p3], sg[p3]).wait()

    def _wait_sc(p6, p3):
        pltpu.make_async_copy(rf[p3], acc_sh.at[dst_v.at[p6]], sc[p3]).wait()

    def _body(c, p6, p3, wait_sc, pf_gat, pf_idx):
        _wait_gat(p6, p3)
        if wait_sc:
            _wait_sc((p6 + 4) % 6, (p3 + 1) % 3)
        if pf_gat:
            _wait_idx((p6 + 1) % 6)
            _issue_gat((p6 + 1) % 6, (p3 + 1) % 3)

        @pl.loop(0, C)
        def _scale(e):
            s = plsc.load_gather(w_v.at[p6], [jnp.full((L,), e, jnp.int32)])
            for k in range(D // L):
                rf[p3][e, pl.ds(k * L, L)] = rf[p3][e, pl.ds(k * L, L)] * s

        pltpu.async_copy(rf[p3], acc_sh.at[dst_v.at[p6]], sc[p3], add=True)
        if pf_idx:
            _issue_idx(c + 3, (p6 + 3) % 6)

    _issue_idx(0, 0)
    _issue_idx(1, 1)
    _issue_idx(2, 2)
    _wait_idx(0)
    _issue_gat(0, 0)
    _body(0, 0, 0, wait_sc=False, pf_gat=True, pf_idx=True)
    _body(1, 1, 1, wait_sc=False, pf_gat=True, pf_idx=True)

    @pl.loop(0, (NCH - 6) // 6)
    def _main(j):
        for u in range(6):
            c = 2 + 6 * j + u
            _body(c, (2 + u) % 6, (2 + u) % 3,
                  wait_sc=True, pf_gat=True, pf_idx=True)

    for u in range(4):
        c = NCH - 4 + u
        _body(c, c % 6, c % 3, wait_sc=True,
              pf_gat=(u < 3), pf_idx=(u == 0))

    _wait_sc((NCH - 2) % 6, (NCH - 2) % 3)
    _wait_sc((NCH - 1) % 6, (NCH - 1) % 3)

    plsc.subcore_barrier()
    pltpu.sync_copy(acc_sh.at[pl.ds(r0, SB)], accp_hbm.at[cid, pl.ds(r0, SB)])

    @pl.when(sid == NS - 1)
    def _dtail():
        t0 = NS * SB
        pltpu.sync_copy(acc_sh.at[pl.ds(t0, ET)], accp_hbm.at[cid, pl.ds(t0, ET)])


_rpass = pl.kernel(
    _rpass_body,
    out_type=jax.ShapeDtypeStruct((NC, N, D), jnp.float32),
    mesh=_MESH,
    compiler_params=_sc_params,
    scratch_types=[
        pltpu.VMEM((6, C), jnp.int32),
        pltpu.VMEM((6, C), jnp.int32),
        pltpu.VMEM((6, C), jnp.float32),
        pltpu.VMEM((C, D), jnp.float32),
        pltpu.VMEM((C, D), jnp.float32),
        pltpu.VMEM((C, D), jnp.float32),
        pltpu.VMEM_SHARED((N, D), jnp.float32),
        pltpu.SemaphoreType.DMA,
        pltpu.SemaphoreType.DMA,
        pltpu.SemaphoreType.DMA,
        pltpu.SemaphoreType.DMA,
        pltpu.SemaphoreType.DMA,
        pltpu.SemaphoreType.DMA,
        pltpu.SemaphoreType.DMA,
        pltpu.SemaphoreType.DMA,
        pltpu.SemaphoreType.DMA,
        pltpu.SemaphoreType.DMA,
        pltpu.SemaphoreType.DMA,
        pltpu.SemaphoreType.DMA,
    ],
)


def _gat_layer(x, src, dst, W, a_src, a_dst, b):
    h, asr, adr, mv = _prep(x, W, a_src.reshape(1, D), a_dst.reshape(1, D))
    wts, denp = _wpass(src, dst, asr, adr, mv)
    accp = _rpass(h, src, dst, wts)
    return _fin(accp, denp, b.reshape(1, D))


def kernel(x, edge_index, W1, a_src1, a_dst1, b1, W2, a_src2, a_dst2, b2):
    loop = jnp.arange(N, dtype=jnp.int32)
    pad = jnp.arange(EPAD - EXT, dtype=jnp.int32) * 5 % N
    src = jnp.concatenate([edge_index[0], loop, pad])
    dst = jnp.concatenate([edge_index[1], loop, pad])
    z = _gat_layer(x, src, dst, W1, a_src1, a_dst1, b1)
    xbar = _gat_layer(z, src, dst, W2, a_src2, a_dst2, b2)
    return xbar, z


# speedup vs baseline: 1.7501x; 1.0083x over previous
import dataclasses
import functools

import jax
import jax.numpy as jnp
from jax import lax
from jax.experimental import pallas as pl
from jax.experimental.pallas import tpu as pltpu
from jax.experimental.pallas import tpu_sc as plsc

N = 10000
E = 320000
D = 128

NC = 2
NS = 16
L = 16
NW = NC * NS

EXT = E + N
EPT = 10368
EPAD = EPT * NW
CW = 576
NCHW = EPT // CW
C = 96
NCH = EPT // C
SB = 624
ET = N - NS * SB


def _prep_body(x_ref, w_ref, asrc_ref, adst_ref, h_ref, as_ref, ad_ref, mv_ref):
    x = x_ref[...]
    w = w_ref[...]
    h = lax.dot_general(x, w, (((1,), (1,)), ((), ())),
                        preferred_element_type=jnp.float32)
    h_ref[...] = h
    asr = lax.dot_general(asrc_ref[...], h, (((1,), (1,)), ((), ())),
                          preferred_element_type=jnp.float32)
    adr = lax.dot_general(adst_ref[...], h, (((1,), (1,)), ((), ())),
                          preferred_element_type=jnp.float32)
    as_ref[...] = asr
    ad_ref[...] = adr
    m = jnp.max(asr) + jnp.max(adr)
    m = jnp.maximum(m, 0.2 * m)
    mv_ref[...] = jnp.full((8, 128), m, jnp.float32)


_prep = pl.pallas_call(
    _prep_body,
    out_shape=[
        jax.ShapeDtypeStruct((N, D), jnp.float32),
        jax.ShapeDtypeStruct((1, N), jnp.float32),
        jax.ShapeDtypeStruct((1, N), jnp.float32),
        jax.ShapeDtypeStruct((8, 128), jnp.float32),
    ],
)


def _fin_body(p_ref, den_ref, b_ref, o_ref):
    num = p_ref[0] + p_ref[1]
    ones = jnp.ones((NW, 1), jnp.float32)
    dcol = lax.dot_general(den_ref[...], ones, (((0,), (0,)), ((), ())),
                           preferred_element_type=jnp.float32)
    o = num / (dcol + 1e-16) + b_ref[...]
    o_ref[...] = jnp.where(o > 0, o, jnp.exp(jnp.minimum(o, 0.0)) - 1.0)


_fin = pl.pallas_call(
    _fin_body,
    out_shape=jax.ShapeDtypeStruct((N, D), jnp.float32),
)


def _finprep_body(p_ref, den_ref, b_ref, w_ref, asrc_ref, adst_ref,
                  z_ref, h_ref, as_ref, ad_ref, mv_ref):
    _fin_body(p_ref, den_ref, b_ref, z_ref)
    _prep_body(z_ref, w_ref, asrc_ref, adst_ref, h_ref, as_ref, ad_ref, mv_ref)


_finprep = pl.pallas_call(
    _finprep_body,
    out_shape=[
        jax.ShapeDtypeStruct((N, D), jnp.float32),
        jax.ShapeDtypeStruct((N, D), jnp.float32),
        jax.ShapeDtypeStruct((1, N), jnp.float32),
        jax.ShapeDtypeStruct((1, N), jnp.float32),
        jax.ShapeDtypeStruct((8, 128), jnp.float32),
    ],
)


_sc_params = pltpu.CompilerParams()
if "needs_layout_passes" in pltpu.CompilerParams.__dataclass_fields__:
    _sc_params = dataclasses.replace(_sc_params, needs_layout_passes=False)

_MESH = plsc.VectorSubcoreMesh(core_axis_name="c", subcore_axis_name="s")


def _wpass_body(src_hbm, dst_hbm, as_hbm, ad_hbm, mv_hbm,
                w_hbm, denp_hbm,
                as_v, ad_v, mv_v, src0_v, src1_v, dst0_v, dst1_v,
                w0_v, w1_v, den_v, si0, si1, so0, so1):
    cid = lax.axis_index("c")
    sid = lax.axis_index("s")
    wid = cid * NS + sid
    srcs, dsts = [src0_v, src1_v], [dst0_v, dst1_v]
    wouts, si, so = [w0_v, w1_v], [si0, si1], [so0, so1]

    pltpu.sync_copy(as_hbm.at[0], as_v)
    pltpu.sync_copy(ad_hbm.at[0], ad_v)
    pltpu.sync_copy(mv_hbm.at[0, pl.ds(0, L)], mv_v)

    zero16 = jnp.zeros((L,), jnp.float32)

    @pl.loop(0, N // L)
    def _zden(r):
        den_v[pl.ds(r * L, L)] = zero16

    mvec = mv_v[...]

    def _issue_idx(c, p):
        base = wid * EPT + c * CW
        pltpu.async_copy(src_hbm.at[pl.ds(base, CW)], srcs[p], si[p])
        pltpu.async_copy(dst_hbm.at[pl.ds(base, CW)], dsts[p], si[p])

    def _wait_idx(p):
        pltpu.make_async_copy(src_hbm.at[pl.ds(0, CW)], srcs[p], si[p]).wait()
        pltpu.make_async_copy(dst_hbm.at[pl.ds(0, CW)], dsts[p], si[p]).wait()

    def _body(c, p, pf_idx, wait_out):
        _wait_idx(p)
        if wait_out:
            pltpu.make_async_copy(w_hbm.at[pl.ds(0, CW)], wouts[p],
                                  so[p]).wait()
        for g in range(CW // L):
            s16 = srcs[p][pl.ds(g * L, L)]
            d16 = dsts[p][pl.ds(g * L, L)]
            av = plsc.load_gather(as_v, [s16])
            dv = plsc.load_gather(ad_v, [d16])
            t = av + dv
            alpha = jnp.maximum(t, 0.2 * t)
            wv = jnp.exp(alpha - mvec)
            eid = wid * EPT + c * CW + g * L + lax.iota(jnp.int32, L)
            wv = jnp.where(eid < EXT, wv, 0.0)
            wouts[p][pl.ds(g * L, L)] = wv
            plsc.addupdate_scatter(den_v, [d16], wv)
        base = wid * EPT + c * CW
        pltpu.async_copy(wouts[p], w_hbm.at[pl.ds(base, CW)], so[p])
        if pf_idx:
            _issue_idx(c + 2, p)

    _issue_idx(0, 0)
    _issue_idx(1, 1)
    _body(0, 0, pf_idx=True, wait_out=False)
    _body(1, 1, pf_idx=True, wait_out=False)

    @pl.loop(0, (NCHW - 4) // 2)
    def _main(j):
        for b in range(2):
            _body(2 + 2 * j + b, b, pf_idx=True, wait_out=True)

    _body(NCHW - 2, 0, pf_idx=False, wait_out=True)
    _body(NCHW - 1, 1, pf_idx=False, wait_out=True)
    pltpu.make_async_copy(w_hbm.at[pl.ds(0, CW)], wouts[0], so[0]).wait()
    pltpu.make_async_copy(w_hbm.at[pl.ds(0, CW)], wouts[1], so[1]).wait()

    pltpu.sync_copy(den_v, denp_hbm.at[wid])


_wpass = pl.kernel(
    _wpass_body,
    out_type=[
        jax.ShapeDtypeStruct((EPAD,), jnp.float32),
        jax.ShapeDtypeStruct((NW, N), jnp.float32),
    ],
    mesh=_MESH,
    compiler_params=_sc_params,
    scratch_types=[
        pltpu.VMEM((N,), jnp.float32),
        pltpu.VMEM((N,), jnp.float32),
        pltpu.VMEM((L,), jnp.float32),
        pltpu.VMEM((CW,), jnp.int32),
        pltpu.VMEM((CW,), jnp.int32),
        pltpu.VMEM((CW,), jnp.int32),
        pltpu.VMEM((CW,), jnp.int32),
        pltpu.VMEM((CW,), jnp.float32),
        pltpu.VMEM((CW,), jnp.float32),
        pltpu.VMEM((N,), jnp.float32),
        pltpu.SemaphoreType.DMA,
        pltpu.SemaphoreType.DMA,
        pltpu.SemaphoreType.DMA,
        pltpu.SemaphoreType.DMA,
    ],
)


def _rpass_body(h_hbm, src_hbm, dst_hbm, w_hbm,
                accp_hbm,
                src_v, dst_v, w_v, rf0_v, rf1_v, rf2_v,
                acc_sh, si0, si1, si2, si3, si4, si5,
                sg0, sg1, sg2, sc0, sc1, sc2):
    cid = lax.axis_index("c")
    sid = lax.axis_index("s")
    wid = cid * NS + sid
    rf = [rf0_v, rf1_v, rf2_v]
    si = [si0, si1, si2, si3, si4, si5]
    sg, sc = [sg0, sg1, sg2], [sc0, sc1, sc2]

    zero16 = jnp.zeros((L,), jnp.float32)

    @pl.loop(0, C)
    def _zrow(r):
        for k in range(D // L):
            rf0_v[r, pl.ds(k * L, L)] = zero16

    r0 = sid * SB

    @pl.loop(0, SB // C)
    def _zacc(j):
        pltpu.sync_copy(rf0_v, acc_sh.at[pl.ds(r0 + j * C, C)])

    zrem = SB % C
    if zrem:
        pltpu.sync_copy(rf0_v.at[pl.ds(0, zrem)],
                        acc_sh.at[pl.ds(r0 + (SB // C) * C, zrem)])

    @pl.when(sid == NS - 1)
    def _ztail():
        pltpu.sync_copy(rf0_v.at[pl.ds(0, ET)], acc_sh.at[pl.ds(NS * SB, ET)])

    plsc.subcore_barrier()

    def _issue_idx(c, p):
        base = wid * EPT + c * C
        pltpu.async_copy(src_hbm.at[pl.ds(base, C)], src_v.at[p], si[p])
        pltpu.async_copy(dst_hbm.at[pl.ds(base, C)], dst_v.at[p], si[p])
        pltpu.async_copy(w_hbm.at[pl.ds(base, C)], w_v.at[p], si[p])

    def _wait_idx(p):
        pltpu.make_async_copy(src_hbm.at[pl.ds(0, C)], src_v.at[p],
                              si[p]).wait()
        pltpu.make_async_copy(dst_hbm.at[pl.ds(0, C)], dst_v.at[p],
                              si[p]).wait()
        pltpu.make_async_copy(w_hbm.at[pl.ds(0, C)], w_v.at[p], si[p]).wait()

    H2 = C // 2

    def _issue_gat(p6, p3):
        pltpu.async_copy(h_hbm.at[src_v.at[p6, pl.ds(0, H2)]],
                         rf[p3].at[pl.ds(0, H2)], sg[p3])
        pltpu.async_copy(h_hbm.at[src_v.at[p6, pl.ds(H2, H2)]],
                         rf[p3].at[pl.ds(H2, H2)], sg[p3])

    def _wait_gat(p6, p3):
        pltpu.make_async_copy(h_hbm.at[src_v.at[p6, pl.ds(0, H2)]],
                              rf[p3].at[pl.ds(0, H2)], sg[p3]).wait()
        pltpu.make_async_copy(h_hbm.at[src_v.at[p6, pl.ds(H2, H2)]],
                              rf[p3].at[pl.ds(H2, H2)], sg[p3]).wait()

    def _wait_sc(p6, p3):
        pltpu.make_async_copy(rf[p3], acc_sh.at[dst_v.at[p6]], sc[p3]).wait()

    def _body(c, p6, p3, wait_sc, pf_gat, pf_idx):
        _wait_gat(p6, p3)
        if wait_sc:
            _wait_sc((p6 + 4) % 6, (p3 + 1) % 3)
        if pf_gat:
            _wait_idx((p6 + 1) % 6)
            _issue_gat((p6 + 1) % 6, (p3 + 1) % 3)

        @pl.loop(0, C)
        def _scale(e):
            s = plsc.load_gather(w_v.at[p6], [jnp.full((L,), e, jnp.int32)])
            for k in range(D // L):
                rf[p3][e, pl.ds(k * L, L)] = rf[p3][e, pl.ds(k * L, L)] * s

        pltpu.async_copy(rf[p3], acc_sh.at[dst_v.at[p6]], sc[p3], add=True)
        if pf_idx:
            _issue_idx(c + 3, (p6 + 3) % 6)

    _issue_idx(0, 0)
    _issue_idx(1, 1)
    _issue_idx(2, 2)
    _wait_idx(0)
    _issue_gat(0, 0)
    _body(0, 0, 0, wait_sc=False, pf_gat=True, pf_idx=True)
    _body(1, 1, 1, wait_sc=False, pf_gat=True, pf_idx=True)

    @pl.loop(0, (NCH - 6) // 6)
    def _main(j):
        for u in range(6):
            c = 2 + 6 * j + u
            _body(c, (2 + u) % 6, (2 + u) % 3,
                  wait_sc=True, pf_gat=True, pf_idx=True)

    for u in range(4):
        c = NCH - 4 + u
        _body(c, c % 6, c % 3, wait_sc=True,
              pf_gat=(u < 3), pf_idx=(u == 0))

    _wait_sc((NCH - 2) % 6, (NCH - 2) % 3)
    _wait_sc((NCH - 1) % 6, (NCH - 1) % 3)

    plsc.subcore_barrier()
    pltpu.sync_copy(acc_sh.at[pl.ds(r0, SB)], accp_hbm.at[cid, pl.ds(r0, SB)])

    @pl.when(sid == NS - 1)
    def _dtail():
        t0 = NS * SB
        pltpu.sync_copy(acc_sh.at[pl.ds(t0, ET)], accp_hbm.at[cid, pl.ds(t0, ET)])


_rpass = pl.kernel(
    _rpass_body,
    out_type=jax.ShapeDtypeStruct((NC, N, D), jnp.float32),
    mesh=_MESH,
    compiler_params=_sc_params,
    scratch_types=[
        pltpu.VMEM((6, C), jnp.int32),
        pltpu.VMEM((6, C), jnp.int32),
        pltpu.VMEM((6, C), jnp.float32),
        pltpu.VMEM((C, D), jnp.float32),
        pltpu.VMEM((C, D), jnp.float32),
        pltpu.VMEM((C, D), jnp.float32),
        pltpu.VMEM_SHARED((N, D), jnp.float32),
        pltpu.SemaphoreType.DMA,
        pltpu.SemaphoreType.DMA,
        pltpu.SemaphoreType.DMA,
        pltpu.SemaphoreType.DMA,
        pltpu.SemaphoreType.DMA,
        pltpu.SemaphoreType.DMA,
        pltpu.SemaphoreType.DMA,
        pltpu.SemaphoreType.DMA,
        pltpu.SemaphoreType.DMA,
        pltpu.SemaphoreType.DMA,
        pltpu.SemaphoreType.DMA,
        pltpu.SemaphoreType.DMA,
    ],
)


def kernel(x, edge_index, W1, a_src1, a_dst1, b1, W2, a_src2, a_dst2, b2):
    loop = jnp.arange(N, dtype=jnp.int32)
    pad = jnp.arange(EPAD - EXT, dtype=jnp.int32) * 5 % N
    src = jnp.concatenate([edge_index[0], loop, pad])
    dst = jnp.concatenate([edge_index[1], loop, pad])

    h1, as1, ad1, mv1 = _prep(x, W1, a_src1.reshape(1, D),
                              a_dst1.reshape(1, D))
    w1, denp1 = _wpass(src, dst, as1, ad1, mv1)
    accp1 = _rpass(h1, src, dst, w1)
    z, h2, as2, ad2, mv2 = _finprep(accp1, denp1, b1.reshape(1, D), W2,
                                    a_src2.reshape(1, D),
                                    a_dst2.reshape(1, D))
    w2, denp2 = _wpass(src, dst, as2, ad2, mv2)
    accp2 = _rpass(h2, src, dst, w2)
    xbar = _fin(accp2, denp2, b2.reshape(1, D))
    return xbar, z


# speedup vs baseline: 1.7528x; 1.0015x over previous
import dataclasses

import jax
import jax.numpy as jnp
from jax import lax
from jax.experimental import pallas as pl
from jax.experimental.pallas import tpu as pltpu
from jax.experimental.pallas import tpu_sc as plsc

N = 10000
E = 320000
D = 128

NC = 2
NS = 16
L = 16
NW = NC * NS

EXT = E + N
EPT = 10368
EPAD = EPT * NW
CW = 576
NCHW = EPT // CW
C = 96
NCH = EPT // C
SB = 624
ET = N - NS * SB


def _prep_body(x_ref, w_ref, asrc_ref, adst_ref, h_ref, as_ref, ad_ref, mv_ref):
    x = x_ref[...]
    w = w_ref[...]
    h = lax.dot_general(x, w, (((1,), (1,)), ((), ())),
                        preferred_element_type=jnp.float32)
    h_ref[...] = h
    asr = lax.dot_general(asrc_ref[...], h, (((1,), (1,)), ((), ())),
                          preferred_element_type=jnp.float32)
    adr = lax.dot_general(adst_ref[...], h, (((1,), (1,)), ((), ())),
                          preferred_element_type=jnp.float32)
    as_ref[...] = asr
    ad_ref[...] = adr
    m = jnp.max(asr) + jnp.max(adr)
    m = jnp.maximum(m, 0.2 * m)
    mv_ref[...] = jnp.full((8, 128), m, jnp.float32)


_prep = pl.pallas_call(
    _prep_body,
    out_shape=[
        jax.ShapeDtypeStruct((N, D), jnp.float32),
        jax.ShapeDtypeStruct((1, N), jnp.float32),
        jax.ShapeDtypeStruct((1, N), jnp.float32),
        jax.ShapeDtypeStruct((8, 128), jnp.float32),
    ],
)


def _fin_body(p_ref, den_ref, b_ref, o_ref):
    num = p_ref[0] + p_ref[1]
    ones = jnp.ones((NW, 1), jnp.float32)
    dcol = lax.dot_general(den_ref[...], ones, (((0,), (0,)), ((), ())),
                           preferred_element_type=jnp.float32)
    o = num / (dcol + 1e-16) + b_ref[...]
    o_ref[...] = jnp.where(o > 0, o, jnp.exp(jnp.minimum(o, 0.0)) - 1.0)


_fin = pl.pallas_call(
    _fin_body,
    out_shape=jax.ShapeDtypeStruct((N, D), jnp.float32),
)


def _finprep_body(p_ref, den_ref, b_ref, w_ref, asrc_ref, adst_ref,
                  z_ref, h_ref, as_ref, ad_ref, mv_ref):
    _fin_body(p_ref, den_ref, b_ref, z_ref)
    _prep_body(z_ref, w_ref, asrc_ref, adst_ref, h_ref, as_ref, ad_ref, mv_ref)


_finprep = pl.pallas_call(
    _finprep_body,
    out_shape=[
        jax.ShapeDtypeStruct((N, D), jnp.float32),
        jax.ShapeDtypeStruct((N, D), jnp.float32),
        jax.ShapeDtypeStruct((1, N), jnp.float32),
        jax.ShapeDtypeStruct((1, N), jnp.float32),
        jax.ShapeDtypeStruct((8, 128), jnp.float32),
    ],
)


_sc_params = pltpu.CompilerParams()
if "needs_layout_passes" in pltpu.CompilerParams.__dataclass_fields__:
    _sc_params = dataclasses.replace(_sc_params, needs_layout_passes=False)

_MESH = plsc.VectorSubcoreMesh(core_axis_name="c", subcore_axis_name="s")


def _wpass_body(src_hbm, dst_hbm, as_hbm, ad_hbm, mv_hbm,
                w_hbm, denp_hbm,
                as_v, ad_v, mv_v, src0_v, src1_v, dst0_v, dst1_v,
                w0_v, w1_v, den_v, si0, si1, so0, so1):
    cid = lax.axis_index("c")
    sid = lax.axis_index("s")
    wid = cid * NS + sid
    srcs, dsts = [src0_v, src1_v], [dst0_v, dst1_v]
    wouts, si, so = [w0_v, w1_v], [si0, si1], [so0, so1]

    pltpu.sync_copy(as_hbm.at[0], as_v)
    pltpu.sync_copy(ad_hbm.at[0], ad_v)
    pltpu.sync_copy(mv_hbm.at[0, pl.ds(0, L)], mv_v)

    zero16 = jnp.zeros((L,), jnp.float32)

    @pl.loop(0, N // L)
    def _zden(r):
        den_v[pl.ds(r * L, L)] = zero16

    mvec = mv_v[...]

    def _issue_idx(c, p):
        base = wid * EPT + c * CW
        pltpu.async_copy(src_hbm.at[pl.ds(base, CW)], srcs[p], si[p])
        pltpu.async_copy(dst_hbm.at[pl.ds(base, CW)], dsts[p], si[p])

    def _wait_idx(p):
        pltpu.make_async_copy(src_hbm.at[pl.ds(0, CW)], srcs[p], si[p]).wait()
        pltpu.make_async_copy(dst_hbm.at[pl.ds(0, CW)], dsts[p], si[p]).wait()

    def _body(c, p, pf_idx, wait_out):
        _wait_idx(p)
        if wait_out:
            pltpu.make_async_copy(w_hbm.at[pl.ds(0, CW)], wouts[p],
                                  so[p]).wait()
        for g in range(CW // L):
            s16 = srcs[p][pl.ds(g * L, L)]
            d16 = dsts[p][pl.ds(g * L, L)]
            av = plsc.load_gather(as_v, [s16])
            dv = plsc.load_gather(ad_v, [d16])
            t = av + dv
            alpha = jnp.maximum(t, 0.2 * t)
            wv = jnp.exp(alpha - mvec)
            eid = wid * EPT + c * CW + g * L + lax.iota(jnp.int32, L)
            wv = jnp.where(eid < EXT, wv, 0.0)
            wouts[p][pl.ds(g * L, L)] = wv
            plsc.addupdate_scatter(den_v, [d16], wv)
        base = wid * EPT + c * CW
        pltpu.async_copy(wouts[p], w_hbm.at[pl.ds(base, CW)], so[p])
        if pf_idx:
            _issue_idx(c + 2, p)

    _issue_idx(0, 0)
    _issue_idx(1, 1)
    _body(0, 0, pf_idx=True, wait_out=False)
    _body(1, 1, pf_idx=True, wait_out=False)

    @pl.loop(0, (NCHW - 4) // 2)
    def _main(j):
        for b in range(2):
            _body(2 + 2 * j + b, b, pf_idx=True, wait_out=True)

    _body(NCHW - 2, 0, pf_idx=False, wait_out=True)
    _body(NCHW - 1, 1, pf_idx=False, wait_out=True)
    pltpu.make_async_copy(w_hbm.at[pl.ds(0, CW)], wouts[0], so[0]).wait()
    pltpu.make_async_copy(w_hbm.at[pl.ds(0, CW)], wouts[1], so[1]).wait()

    pltpu.sync_copy(den_v, denp_hbm.at[wid])


_wpass = pl.kernel(
    _wpass_body,
    out_type=[
        jax.ShapeDtypeStruct((EPAD,), jnp.float32),
        jax.ShapeDtypeStruct((NW, N), jnp.float32),
    ],
    mesh=_MESH,
    compiler_params=_sc_params,
    scratch_types=[
        pltpu.VMEM((N,), jnp.float32),
        pltpu.VMEM((N,), jnp.float32),
        pltpu.VMEM((L,), jnp.float32),
        pltpu.VMEM((CW,), jnp.int32),
        pltpu.VMEM((CW,), jnp.int32),
        pltpu.VMEM((CW,), jnp.int32),
        pltpu.VMEM((CW,), jnp.int32),
        pltpu.VMEM((CW,), jnp.float32),
        pltpu.VMEM((CW,), jnp.float32),
        pltpu.VMEM((N,), jnp.float32),
        pltpu.SemaphoreType.DMA,
        pltpu.SemaphoreType.DMA,
        pltpu.SemaphoreType.DMA,
        pltpu.SemaphoreType.DMA,
    ],
)


def _rpass_body(h_hbm, src_hbm, dst_hbm, w_hbm,
                accp_hbm,
                src_v, dst_v, w_v, rf0_v, rf1_v, rf2_v,
                acc_sh, si0, si1, si2, si3, si4, si5,
                sg0, sg1, sg2, sc0, sc1, sc2):
    cid = lax.axis_index("c")
    sid = lax.axis_index("s")
    wid = cid * NS + sid
    rf = [rf0_v, rf1_v, rf2_v]
    si = [si0, si1, si2, si3, si4, si5]
    sg, sc = [sg0, sg1, sg2], [sc0, sc1, sc2]

    zero16 = jnp.zeros((L,), jnp.float32)

    @pl.loop(0, C)
    def _zrow(r):
        for k in range(D // L):
            rf0_v[r, pl.ds(k * L, L)] = zero16

    r0 = sid * SB

    @pl.loop(0, SB // C)
    def _zacc(j):
        pltpu.sync_copy(rf0_v, acc_sh.at[pl.ds(r0 + j * C, C)])

    zrem = SB % C
    if zrem:
        pltpu.sync_copy(rf0_v.at[pl.ds(0, zrem)],
                        acc_sh.at[pl.ds(r0 + (SB // C) * C, zrem)])

    @pl.when(sid == NS - 1)
    def _ztail():
        pltpu.sync_copy(rf0_v.at[pl.ds(0, ET)], acc_sh.at[pl.ds(NS * SB, ET)])

    plsc.subcore_barrier()

    def _issue_idx(c, p):
        base = wid * EPT + c * C
        pltpu.async_copy(src_hbm.at[pl.ds(base, C)], src_v.at[p], si[p])
        pltpu.async_copy(dst_hbm.at[pl.ds(base, C)], dst_v.at[p], si[p])
        pltpu.async_copy(w_hbm.at[pl.ds(base, C)], w_v.at[p], si[p])

    def _wait_idx(p):
        pltpu.make_async_copy(src_hbm.at[pl.ds(0, C)], src_v.at[p],
                              si[p]).wait()
        pltpu.make_async_copy(dst_hbm.at[pl.ds(0, C)], dst_v.at[p],
                              si[p]).wait()
        pltpu.make_async_copy(w_hbm.at[pl.ds(0, C)], w_v.at[p], si[p]).wait()

    H2 = C // 2

    def _issue_gat(p6, p3):
        pltpu.async_copy(h_hbm.at[src_v.at[p6, pl.ds(0, H2)]],
                         rf[p3].at[pl.ds(0, H2)], sg[p3])
        pltpu.async_copy(h_hbm.at[src_v.at[p6, pl.ds(H2, H2)]],
                         rf[p3].at[pl.ds(H2, H2)], sg[p3])

    def _wait_gat(p6, p3):
        pltpu.make_async_copy(h_hbm.at[src_v.at[p6, pl.ds(0, H2)]],
                              rf[p3].at[pl.ds(0, H2)], sg[p3]).wait()
        pltpu.make_async_copy(h_hbm.at[src_v.at[p6, pl.ds(H2, H2)]],
                              rf[p3].at[pl.ds(H2, H2)], sg[p3]).wait()

    def _wait_sc(p6, p3):
        pltpu.make_async_copy(rf[p3], acc_sh.at[dst_v.at[p6]], sc[p3]).wait()

    def _body(c, p6, p3, wait_sc, pf_gat, pf_idx):
        _wait_gat(p6, p3)
        if wait_sc:
            _wait_sc((p6 + 4) % 6, (p3 + 1) % 3)
        if pf_gat:
            _wait_idx((p6 + 1) % 6)
            _issue_gat((p6 + 1) % 6, (p3 + 1) % 3)

        @pl.loop(0, C)
        def _scale(e):
            s = plsc.load_gather(w_v.at[p6], [jnp.full((L,), e, jnp.int32)])
            for k in range(D // L):
                rf[p3][e, pl.ds(k * L, L)] = rf[p3][e, pl.ds(k * L, L)] * s

        pltpu.async_copy(rf[p3], acc_sh.at[dst_v.at[p6]], sc[p3], add=True)
        if pf_idx:
            _issue_idx(c + 3, (p6 + 3) % 6)

    _issue_idx(0, 0)
    _issue_idx(1, 1)
    _issue_idx(2, 2)
    _wait_idx(0)
    _issue_gat(0, 0)
    _body(0, 0, 0, wait_sc=False, pf_gat=True, pf_idx=True)
    _body(1, 1, 1, wait_sc=False, pf_gat=True, pf_idx=True)

    @pl.loop(0, (NCH - 6) // 6)
    def _main(j):
        for u in range(6):
            c = 2 + 6 * j + u
            _body(c, (2 + u) % 6, (2 + u) % 3,
                  wait_sc=True, pf_gat=True, pf_idx=True)

    for u in range(4):
        c = NCH - 4 + u
        _body(c, c % 6, c % 3, wait_sc=True,
              pf_gat=(u < 3), pf_idx=(u == 0))

    _wait_sc((NCH - 2) % 6, (NCH - 2) % 3)
    _wait_sc((NCH - 1) % 6, (NCH - 1) % 3)

    plsc.subcore_barrier()
    pltpu.sync_copy(acc_sh.at[pl.ds(r0, SB)], accp_hbm.at[cid, pl.ds(r0, SB)])

    @pl.when(sid == NS - 1)
    def _dtail():
        t0 = NS * SB
        pltpu.sync_copy(acc_sh.at[pl.ds(t0, ET)], accp_hbm.at[cid, pl.ds(t0, ET)])


_rpass = pl.kernel(
    _rpass_body,
    out_type=jax.ShapeDtypeStruct((NC, N, D), jnp.float32),
    mesh=_MESH,
    compiler_params=_sc_params,
    scratch_types=[
        pltpu.VMEM((6, C), jnp.int32),
        pltpu.VMEM((6, C), jnp.int32),
        pltpu.VMEM((6, C), jnp.float32),
        pltpu.VMEM((C, D), jnp.float32),
        pltpu.VMEM((C, D), jnp.float32),
        pltpu.VMEM((C, D), jnp.float32),
        pltpu.VMEM_SHARED((N, D), jnp.float32),
        pltpu.SemaphoreType.DMA,
        pltpu.SemaphoreType.DMA,
        pltpu.SemaphoreType.DMA,
        pltpu.SemaphoreType.DMA,
        pltpu.SemaphoreType.DMA,
        pltpu.SemaphoreType.DMA,
        pltpu.SemaphoreType.DMA,
        pltpu.SemaphoreType.DMA,
        pltpu.SemaphoreType.DMA,
        pltpu.SemaphoreType.DMA,
        pltpu.SemaphoreType.DMA,
        pltpu.SemaphoreType.DMA,
    ],
)


def kernel(x, edge_index, W1, a_src1, a_dst1, b1, W2, a_src2, a_dst2, b2):
    loop = jnp.arange(N, dtype=jnp.int32)
    pad = jnp.arange(EPAD - EXT, dtype=jnp.int32) * 5 % N
    src = jnp.concatenate([edge_index[0], loop, pad])
    dst = jnp.concatenate([edge_index[1], loop, pad])

    h1, as1, ad1, mv1 = _prep(x, W1, a_src1.reshape(1, D),
                              a_dst1.reshape(1, D))
    w1, denp1 = _wpass(src, dst, as1, ad1, mv1)
    accp1 = _rpass(h1, src, dst, w1)
    z, h2, as2, ad2, mv2 = _finprep(accp1, denp1, b1.reshape(1, D), W2,
                                    a_src2.reshape(1, D),
                                    a_dst2.reshape(1, D))
    w2, denp2 = _wpass(src, dst, as2, ad2, mv2)
    accp2 = _rpass(h2, src, dst, w2)
    xbar = _fin(accp2, denp2, b2.reshape(1, D))
    return xbar, z
